# Initial kernel scaffold; baseline (speedup 1.0000x reference)
import jax
import jax.numpy as jnp
from jax.experimental import pallas as pl


def kernel(token_id, embed_weight):
    raise NotImplementedError("write your pallas kernel here")


# speedup vs baseline: 1.8392x; 1.8392x over previous
import functools

import jax
import jax.numpy as jnp
from jax import lax
from jax.experimental import pallas as pl
from jax.experimental.pallas import tpu as pltpu
from jax.experimental.pallas import tpu_sc as plsc

VOCAB = 1000000
EMBED_DIM = 64
BATCH = 16384
HIST = 50

NC = 2
NS = 16
NW = NC * NS

B_FLAT = BATCH * HIST
B_PER_W = B_FLAT // NW
CHUNK = 128
NCH = B_PER_W // CHUNK

_mesh = plsc.VectorSubcoreMesh(core_axis_name="c", subcore_axis_name="s")


@functools.partial(
    pl.kernel,
    mesh=_mesh,
    compiler_params=pltpu.CompilerParams(use_tc_tiling_on_sc=False),
    out_type=jax.ShapeDtypeStruct((B_FLAT, EMBED_DIM), jnp.float32),
    scratch_types=[
        pltpu.VMEM((NCH, CHUNK), jnp.int32),
        pltpu.VMEM((2, CHUNK, EMBED_DIM), jnp.float32),
        pltpu.SemaphoreType.DMA,
        pltpu.SemaphoreType.DMA,
    ],
)
def _gather_kernel(tok_hbm, table_hbm, out_hbm, idx_v, rows_v, sem0, sem1):
    wid = lax.axis_index("s") * NC + lax.axis_index("c")
    wbase = wid * B_PER_W
    sems = (sem0, sem1)

    pltpu.sync_copy(tok_hbm.at[wid], idx_v)

    def _start_gather(g, b):
        pltpu.async_copy(table_hbm.at[idx_v.at[g]], rows_v.at[b], sems[b])

    def _wait_gather(g, b):
        pltpu.make_async_copy(table_hbm.at[idx_v.at[g]], rows_v.at[b], sems[b]).wait()

    def _writeback(g, b):
        pltpu.sync_copy(rows_v.at[b], out_hbm.at[pl.ds(wbase + g * CHUNK, CHUNK)])

    for b in range(2):
        _start_gather(b, b)

    def _body(i, carry):
        g0 = 2 * i
        for b in range(2):
            g = g0 + b
            _wait_gather(g, b)
            _writeback(g, b)
            _start_gather(g + 2, b)
        return carry

    lax.fori_loop(0, NCH // 2 - 1, _body, 0)

    for b in range(2):
        g = NCH - 2 + b
        _wait_gather(g, b)
        _writeback(g, b)


def kernel(token_id, embed_weight):
    tok = token_id.reshape(NW, NCH, CHUNK)
    out = _gather_kernel(tok, embed_weight)
    return out.reshape(BATCH, HIST, EMBED_DIM)


# speedup vs baseline: 1.8556x; 1.0089x over previous
import functools

import jax
import jax.numpy as jnp
from jax import lax
from jax.experimental import pallas as pl
from jax.experimental.pallas import tpu as pltpu
from jax.experimental.pallas import tpu_sc as plsc

VOCAB = 1000000
EMBED_DIM = 64
BATCH = 16384
HIST = 50

NC = 2
NS = 16
NW = NC * NS

B_FLAT = BATCH * HIST
B_PER_W = B_FLAT // NW
CHUNK = 512
NCH = B_PER_W // CHUNK

_mesh = plsc.VectorSubcoreMesh(core_axis_name="c", subcore_axis_name="s")


@functools.partial(
    pl.kernel,
    mesh=_mesh,
    compiler_params=pltpu.CompilerParams(use_tc_tiling_on_sc=False),
    out_type=jax.ShapeDtypeStruct((B_FLAT, EMBED_DIM), jnp.float32),
    scratch_types=[
        pltpu.VMEM((NCH, CHUNK), jnp.int32),
        pltpu.VMEM((2, CHUNK, EMBED_DIM), jnp.float32),
        pltpu.SemaphoreType.DMA,
        pltpu.SemaphoreType.DMA,
    ],
)
def _gather_kernel(tok_hbm, table_hbm, out_hbm, idx_v, rows_v, sem0, sem1):
    wid = lax.axis_index("s") * NC + lax.axis_index("c")
    wbase = wid * B_PER_W
    sems = (sem0, sem1)

    pltpu.sync_copy(tok_hbm.at[wid], idx_v)

    def _start_gather(g, b):
        pltpu.async_copy(table_hbm.at[idx_v.at[g]], rows_v.at[b], sems[b])

    def _wait_gather(g, b):
        pltpu.make_async_copy(table_hbm.at[idx_v.at[g]], rows_v.at[b], sems[b]).wait()

    def _writeback(g, b):
        pltpu.sync_copy(rows_v.at[b], out_hbm.at[pl.ds(wbase + g * CHUNK, CHUNK)])

    for b in range(2):
        _start_gather(b, b)

    def _body(i, carry):
        g0 = 2 * i
        for b in range(2):
            g = g0 + b
            _wait_gather(g, b)
            _writeback(g, b)
            _start_gather(g + 2, b)
        return carry

    lax.fori_loop(0, NCH // 2 - 1, _body, 0)

    for b in range(2):
        g = NCH - 2 + b
        _wait_gather(g, b)
        _writeback(g, b)


def kernel(token_id, embed_weight):
    tok = token_id.reshape(NW, NCH, CHUNK)
    out = _gather_kernel(tok, embed_weight)
    return out.reshape(BATCH, HIST, EMBED_DIM)


# speedup vs baseline: 1.8785x; 1.0124x over previous
import functools

import jax
import jax.numpy as jnp
from jax import lax
from jax.experimental import pallas as pl
from jax.experimental.pallas import tpu as pltpu
from jax.experimental.pallas import tpu_sc as plsc

VOCAB = 1000000
EMBED_DIM = 64
BATCH = 16384
HIST = 50

NC = 2
NS = 16
NW = NC * NS

B_PER_W = BATCH // NW

_mesh = plsc.VectorSubcoreMesh(core_axis_name="c", subcore_axis_name="s")


@functools.partial(
    pl.kernel,
    mesh=_mesh,
    compiler_params=pltpu.CompilerParams(use_tc_tiling_on_sc=False),
    out_type=jax.ShapeDtypeStruct((BATCH, HIST, EMBED_DIM), jnp.float32),
    scratch_types=[
        pltpu.VMEM((HIST, B_PER_W), jnp.int32),
        pltpu.VMEM((2, B_PER_W, EMBED_DIM), jnp.float32),
        pltpu.SemaphoreType.DMA,
        pltpu.SemaphoreType.DMA,
    ],
)
def _gather_kernel(tok_hbm, table_hbm, out_hbm, idx_v, rows_v, sem0, sem1):
    wid = lax.axis_index("s") * NC + lax.axis_index("c")
    base = wid * B_PER_W
    sems = (sem0, sem1)

    pltpu.sync_copy(tok_hbm.at[:, pl.ds(base, B_PER_W)], idx_v)

    def _start_gather(h, b):
        pltpu.async_copy(table_hbm.at[idx_v.at[h]], rows_v.at[b], sems[b])

    def _wait_gather(h, b):
        pltpu.make_async_copy(table_hbm.at[idx_v.at[h]], rows_v.at[b], sems[b]).wait()

    def _writeback(h, b):
        pltpu.sync_copy(rows_v.at[b], out_hbm.at[pl.ds(base, B_PER_W), h])

    for b in range(2):
        _start_gather(b, b)

    def _body(i, carry):
        h0 = 2 * i
        for b in range(2):
            h = h0 + b
            _wait_gather(h, b)
            _writeback(h, b)
            _start_gather(h + 2, b)
        return carry

    lax.fori_loop(0, HIST // 2 - 1, _body, 0)

    for b in range(2):
        h = HIST - 2 + b
        _wait_gather(h, b)
        _writeback(h, b)


def kernel(token_id, embed_weight):
    return _gather_kernel(token_id.T, embed_weight)
